# baseline (speedup 1.0000x reference)
import functools

import jax
import jax.numpy as jnp
from jax import lax
from jax.experimental import pallas as pl
from jax.experimental.pallas import tpu as pltpu
from jax.experimental.pallas import tpu_sc as plsc

D_MODEL = 64
SCALE = 8.0
NC, NS, L = 2, 16, 16
NW = NC * NS


@functools.partial(jax.jit, static_argnames=("b_per_w", "chunk", "n_chunks"))
def _sc_embed(tflat, lut, b_per_w, chunk, n_chunks):
    B = tflat.shape[0]
    mesh = plsc.VectorSubcoreMesh(core_axis_name="c", subcore_axis_name="s")

    @functools.partial(
        pl.kernel,
        out_type=jax.ShapeDtypeStruct((B, D_MODEL), jnp.float32),
        mesh=mesh,
        scratch_types=[
            pltpu.VMEM((b_per_w,), jnp.int32),
            pltpu.VMEM((chunk, D_MODEL), jnp.float32),
            pltpu.SemaphoreType.DMA,
        ],
        compiler_params=pltpu.CompilerParams(use_tc_tiling_on_sc=False),
    )
    def body(text_hbm, lut_hbm, out_hbm, idx_v, rows_v, sem):
        wid = lax.axis_index("s") * NC + lax.axis_index("c")
        base = wid * b_per_w
        pltpu.sync_copy(text_hbm.at[pl.ds(base, b_per_w)], idx_v)

        def chunk_body(c, carry):
            off = pl.multiple_of(c * chunk, 8)
            pltpu.async_copy(
                lut_hbm.at[idx_v.at[pl.ds(off, chunk)]], rows_v, sem
            ).wait()

            def scale_body(i, carry2):
                for j in range(D_MODEL // L):
                    sl = pl.ds(j * L, L)
                    rows_v[i, sl] = rows_v[i, sl] * SCALE
                return carry2

            lax.fori_loop(0, chunk, scale_body, 0)
            pltpu.sync_copy(rows_v, out_hbm.at[pl.ds(base + off, chunk)])
            return carry

        lax.fori_loop(0, n_chunks, chunk_body, 0)

    return body(tflat, lut)


def kernel(text, lut):
    b, s = text.shape
    B = b * s
    tflat = text.reshape(B).astype(jnp.int32)
    b_per_w = B // NW
    chunk = 800
    n_chunks = b_per_w // chunk
    out = _sc_embed(tflat, lut, b_per_w, chunk, n_chunks)
    return out.reshape(b, s, D_MODEL)


# speedup vs baseline: 1.1042x; 1.1042x over previous
import functools

import jax
import jax.numpy as jnp
from jax import lax
from jax.experimental import pallas as pl
from jax.experimental.pallas import tpu as pltpu
from jax.experimental.pallas import tpu_sc as plsc

D_MODEL = 64
SCALE = 8.0
NC, NS, L = 2, 16, 16
NW = NC * NS


@functools.partial(jax.jit, static_argnames=("b_per_w", "chunk", "n_chunks"))
def _sc_embed(tflat, lut, b_per_w, chunk, n_chunks):
    B = tflat.shape[0]
    mesh = plsc.VectorSubcoreMesh(core_axis_name="c", subcore_axis_name="s")

    @functools.partial(
        pl.kernel,
        out_type=jax.ShapeDtypeStruct((B, D_MODEL), jnp.float32),
        mesh=mesh,
        scratch_types=[
            pltpu.VMEM((b_per_w,), jnp.int32),
            pltpu.VMEM((chunk, D_MODEL), jnp.float32),
            pltpu.VMEM((chunk, D_MODEL), jnp.float32),
            pltpu.VMEM((chunk, D_MODEL), jnp.float32),
            pltpu.VMEM((chunk, D_MODEL), jnp.float32),
            pltpu.SemaphoreType.DMA,
            pltpu.SemaphoreType.DMA,
            pltpu.SemaphoreType.DMA,
            pltpu.SemaphoreType.DMA,
        ],
        compiler_params=pltpu.CompilerParams(use_tc_tiling_on_sc=False),
    )
    def body(text_hbm, lut_hbm, out_hbm, idx_v, in0, in1, ou0, ou1,
             gs0, gs1, ss0, ss1):
        wid = lax.axis_index("s") * NC + lax.axis_index("c")
        base = wid * b_per_w
        pltpu.sync_copy(text_hbm.at[pl.ds(base, b_per_w)], idx_v)

        def gather_start(c, ibuf, gsem):
            off = pl.multiple_of(c * chunk, 8)
            pltpu.async_copy(lut_hbm.at[idx_v.at[pl.ds(off, chunk)]], ibuf, gsem)

        def gather_wait(ibuf, gsem):
            pltpu.make_async_copy(
                lut_hbm.at[idx_v.at[pl.ds(0, chunk)]], ibuf, gsem
            ).wait()

        def store_start(c, obuf, ssem):
            off = pl.multiple_of(c * chunk, 8)
            pltpu.async_copy(obuf, out_hbm.at[pl.ds(base + off, chunk)], ssem)

        def store_wait(obuf, ssem):
            pltpu.make_async_copy(
                obuf, out_hbm.at[pl.ds(base, chunk)], ssem
            ).wait()

        def scale(ibuf, obuf):
            @plsc.parallel_loop(0, chunk, step=2, unroll=4)
            def _(i):
                for r in range(2):
                    for j in range(D_MODEL // L):
                        sl = pl.ds(j * L, L)
                        obuf[i + r, sl] = ibuf[i + r, sl] * SCALE

        gather_start(0, in0, gs0)
        gather_start(1, in1, gs1)

        def pair_body(g, carry):
            c0 = 2 * g

            def half(c, ibuf, obuf, gsem, ssem):
                gather_wait(ibuf, gsem)

                @pl.when(g > 0)
                def _():
                    store_wait(obuf, ssem)

                scale(ibuf, obuf)

                @pl.when(c + 2 < n_chunks)
                def _():
                    gather_start(c + 2, ibuf, gsem)

                store_start(c, obuf, ssem)

            half(c0, in0, ou0, gs0, ss0)
            half(c0 + 1, in1, ou1, gs1, ss1)
            return carry

        lax.fori_loop(0, n_chunks // 2, pair_body, 0)
        store_wait(ou0, ss0)
        store_wait(ou1, ss1)

    return body(tflat, lut)


def kernel(text, lut):
    b, s = text.shape
    B = b * s
    tflat = text.reshape(B).astype(jnp.int32)
    b_per_w = B // NW
    chunk = 400
    n_chunks = b_per_w // chunk
    out = _sc_embed(tflat, lut, b_per_w, chunk, n_chunks)
    return out.reshape(b, s, D_MODEL)
